# Initial kernel scaffold; baseline (speedup 1.0000x reference)
import functools

import jax
import jax.numpy as jnp
from jax import lax
from jax.experimental import pallas as pl
from jax.experimental.pallas import tpu as pltpu
from jax.experimental.pallas import tpu_sc as plsc

NC = 2
NS = 16
L = 16
CH = 128


def _coeff_body(r_ref, a_ref, wr0_ref, wr1_ref, wattr_ref, o_ref):
    r = r_ref[...]
    h = jax.nn.swish(jnp.dot(r, wr0_ref[...], preferred_element_type=jnp.float32))
    rad = jnp.dot(h, wr1_ref[...], preferred_element_type=jnp.float32)
    am = lax.dot_general(a_ref[...], wattr_ref[...], (((1,), (1,)), ((), ())),
                         preferred_element_type=jnp.float32)
    o_ref[...] = rad * am


def _coeff(radial, attrs, W_r0, W_r1, W_attr, be):
    ep, r_dim = radial.shape
    de = attrs.shape[1]
    d = W_r1.shape[1]
    grid = ep // be
    return pl.pallas_call(
        _coeff_body,
        grid=(grid,),
        in_specs=[
            pl.BlockSpec((be, r_dim), lambda i: (i, 0)),
            pl.BlockSpec((be, de), lambda i: (i, 0)),
            pl.BlockSpec((r_dim, r_dim), lambda i: (0, 0)),
            pl.BlockSpec((r_dim, d), lambda i: (0, 0)),
            pl.BlockSpec((d, de), lambda i: (0, 0)),
        ],
        out_specs=pl.BlockSpec((be, d), lambda i: (i, 0)),
        out_shape=jax.ShapeDtypeStruct((ep, d), jnp.float32),
    )(radial, attrs, W_r0, W_r1, W_attr)


def _matmul_body(x_ref, w_ref, o_ref):
    o_ref[...] = jnp.dot(x_ref[...], w_ref[...], preferred_element_type=jnp.float32)


def _in_linear(nf, W_in, bn):
    n, d = nf.shape
    return pl.pallas_call(
        _matmul_body,
        grid=(n // bn,),
        in_specs=[
            pl.BlockSpec((bn, d), lambda i: (i, 0)),
            pl.BlockSpec((d, d), lambda i: (0, 0)),
        ],
        out_specs=pl.BlockSpec((bn, d), lambda i: (i, 0)),
        out_shape=jax.ShapeDtypeStruct((n, d), jnp.float32),
    )(nf, W_in)


def _make_sc_conv(n, d, ep):
    per_tile = ep // (NC * NS)
    n_chunks = per_tile // CH
    rows_per_sid = n // NS
    mesh = plsc.VectorSubcoreMesh(core_axis_name="c", subcore_axis_name="s")

    @functools.partial(
        pl.kernel,
        out_type=jax.ShapeDtypeStruct((NC, n, d), jnp.float32),
        mesh=mesh,
        scratch_types=[
            pltpu.VMEM((CH,), jnp.int32),
            pltpu.VMEM((CH,), jnp.int32),
            pltpu.VMEM((CH, d), jnp.float32),
            pltpu.VMEM((CH, d), jnp.float32),
            pltpu.VMEM_SHARED((n, d), jnp.float32),
            pltpu.SemaphoreType.DMA,
        ],
    )
    def sc_conv(x_hbm, coeff_hbm, send_hbm, recv_hbm, zeros_hbm, out_hbm,
                sidx_v, ridx_v, rows_v, coef_v, agg_sh, sem):
        cid = lax.axis_index("c")
        sid = lax.axis_index("s")
        wid = cid * NS + sid

        pltpu.sync_copy(zeros_hbm.at[pl.ds(sid * rows_per_sid, rows_per_sid)],
                        agg_sh.at[pl.ds(sid * rows_per_sid, rows_per_sid)])
        plsc.subcore_barrier()

        base = wid * per_tile

        def chunk(i, carry):
            off = base + i * CH
            pltpu.sync_copy(send_hbm.at[pl.ds(off, CH)], sidx_v)
            pltpu.sync_copy(recv_hbm.at[pl.ds(off, CH)], ridx_v)
            pltpu.sync_copy(coeff_hbm.at[pl.ds(off, CH), :], coef_v)
            pltpu.async_copy(x_hbm.at[sidx_v], rows_v, sem).wait()

            def row(j, c2):
                for k in range(d // L):
                    sl = pl.ds(k * L, L)
                    rows_v[j, sl] = rows_v[j, sl] * coef_v[j, sl]
                return c2
            lax.fori_loop(0, CH, row, 0)

            pltpu.sync_copy(rows_v, agg_sh.at[ridx_v], add=True)
            return carry

        lax.fori_loop(0, n_chunks, chunk, 0)
        plsc.subcore_barrier()

        pltpu.sync_copy(agg_sh.at[pl.ds(sid * rows_per_sid, rows_per_sid)],
                        out_hbm.at[cid, pl.ds(sid * rows_per_sid, rows_per_sid)])

    return sc_conv


def _out_body(p_ref, nf_ref, sp_ref, wout_ref, wsc_ref, o_ref, *, inv_sqrt_neigh, s):
    agg = (p_ref[0] + p_ref[1]) * inv_sqrt_neigh
    out = jnp.dot(agg, wout_ref[...], preferred_element_type=jnp.float32)
    nf = nf_ref[...]
    sp = sp_ref[...]
    sc = jnp.zeros_like(out)
    for k in range(s):
        cand = jnp.dot(nf, wsc_ref[k], preferred_element_type=jnp.float32)
        sc = jnp.where(sp == k, cand, sc)
    z = out + sc
    o_ref[...] = z * jax.nn.sigmoid(z)


def _finalize(partials, nf, species2d, W_out, W_sc, inv_sqrt_neigh, bn):
    n, d = nf.shape
    s = W_sc.shape[0]
    body = functools.partial(_out_body, inv_sqrt_neigh=inv_sqrt_neigh, s=s)
    return pl.pallas_call(
        body,
        grid=(n // bn,),
        in_specs=[
            pl.BlockSpec((NC, bn, d), lambda i: (0, i, 0)),
            pl.BlockSpec((bn, d), lambda i: (i, 0)),
            pl.BlockSpec((bn, 1), lambda i: (i, 0)),
            pl.BlockSpec((d, d), lambda i: (0, 0)),
            pl.BlockSpec((s, d, d), lambda i: (0, 0, 0)),
        ],
        out_specs=pl.BlockSpec((bn, d), lambda i: (i, 0)),
        out_shape=jax.ShapeDtypeStruct((n, d), jnp.float32),
    )(partials, nf, species2d, W_out, W_sc)


def kernel(node_features, edge_attributes, radial_embeddings, senders,
           receivers, species, W_in, W_r0, W_r1, W_attr, W_out, W_sc):
    n, d = node_features.shape
    e = senders.shape[0]
    avg_neigh = 32.0

    gran = NC * NS * CH
    ep = ((e + gran - 1) // gran) * gran
    pad = ep - e
    if pad:
        radial_p = jnp.pad(radial_embeddings, ((0, pad), (0, 0)))
        attrs_p = jnp.pad(edge_attributes, ((0, pad), (0, 0)))
        send_p = jnp.pad(senders.astype(jnp.int32), (0, pad))
        recv_p = jnp.pad(receivers.astype(jnp.int32), (0, pad))
    else:
        radial_p, attrs_p = radial_embeddings, edge_attributes
        send_p = senders.astype(jnp.int32)
        recv_p = receivers.astype(jnp.int32)

    coeff = _coeff(radial_p, attrs_p, W_r0, W_r1, W_attr, be=2048)
    x = _in_linear(node_features, W_in, bn=1000)

    zeros = jnp.zeros((n, d), jnp.float32)
    sc_conv = _make_sc_conv(n, d, ep)
    partials = sc_conv(x, coeff, send_p, recv_p, zeros)

    species2d = species.astype(jnp.int32).reshape(n, 1)
    return _finalize(partials, node_features, species2d, W_out, W_sc,
                     1.0 / (avg_neigh ** 0.5), bn=1000)


# speedup vs baseline: 1.7894x; 1.7894x over previous
import functools

import jax
import jax.numpy as jnp
from jax import lax
from jax.experimental import pallas as pl
from jax.experimental.pallas import tpu as pltpu
from jax.experimental.pallas import tpu_sc as plsc

NC = 2
NS = 16
L = 16
CH = 128


def _coeff_body(r_ref, a_ref, wr0_ref, wr1_ref, wattr_ref, o_ref):
    r = r_ref[...]
    h = jax.nn.swish(jnp.dot(r, wr0_ref[...], preferred_element_type=jnp.float32))
    rad = jnp.dot(h, wr1_ref[...], preferred_element_type=jnp.float32)
    am = lax.dot_general(a_ref[...], wattr_ref[...], (((1,), (1,)), ((), ())),
                         preferred_element_type=jnp.float32)
    o_ref[...] = rad * am


def _coeff(radial, attrs, W_r0, W_r1, W_attr, be):
    ep, r_dim = radial.shape
    de = attrs.shape[1]
    d = W_r1.shape[1]
    grid = ep // be
    return pl.pallas_call(
        _coeff_body,
        grid=(grid,),
        in_specs=[
            pl.BlockSpec((be, r_dim), lambda i: (i, 0)),
            pl.BlockSpec((be, de), lambda i: (i, 0)),
            pl.BlockSpec((r_dim, r_dim), lambda i: (0, 0)),
            pl.BlockSpec((r_dim, d), lambda i: (0, 0)),
            pl.BlockSpec((d, de), lambda i: (0, 0)),
        ],
        out_specs=pl.BlockSpec((be, d), lambda i: (i, 0)),
        out_shape=jax.ShapeDtypeStruct((ep, d), jnp.float32),
    )(radial, attrs, W_r0, W_r1, W_attr)


def _matmul_body(x_ref, w_ref, o_ref):
    o_ref[...] = jnp.dot(x_ref[...], w_ref[...], preferred_element_type=jnp.float32)


def _in_linear(nf, W_in, bn):
    n, d = nf.shape
    return pl.pallas_call(
        _matmul_body,
        grid=(n // bn,),
        in_specs=[
            pl.BlockSpec((bn, d), lambda i: (i, 0)),
            pl.BlockSpec((d, d), lambda i: (0, 0)),
        ],
        out_specs=pl.BlockSpec((bn, d), lambda i: (i, 0)),
        out_shape=jax.ShapeDtypeStruct((n, d), jnp.float32),
    )(nf, W_in)


def _make_sc_conv(n, d, ep):
    per_tile = ep // (NC * NS)
    n_chunks = per_tile // CH
    rows_per_sid = n // NS
    mesh = plsc.VectorSubcoreMesh(core_axis_name="c", subcore_axis_name="s")

    @functools.partial(
        pl.kernel,
        out_type=jax.ShapeDtypeStruct((NC, n, d), jnp.float32),
        mesh=mesh,
        scratch_types=[
            pltpu.VMEM((CH,), jnp.int32),
            pltpu.VMEM((CH,), jnp.int32),
            pltpu.VMEM((CH, d), jnp.float32),
            pltpu.VMEM((CH, d), jnp.float32),
            pltpu.VMEM_SHARED((n, d), jnp.float32),
            pltpu.SemaphoreType.DMA,
        ],
    )
    def sc_conv(x_hbm, coeff_hbm, send_hbm, recv_hbm, zeros_hbm, out_hbm,
                sidx_v, ridx_v, rows_v, coef_v, agg_sh, sem):
        cid = lax.axis_index("c")
        sid = lax.axis_index("s")
        wid = cid * NS + sid

        pltpu.sync_copy(zeros_hbm.at[pl.ds(sid * rows_per_sid, rows_per_sid)],
                        agg_sh.at[pl.ds(sid * rows_per_sid, rows_per_sid)])
        plsc.subcore_barrier()

        base = wid * per_tile

        def chunk(i, carry):
            off = base + i * CH
            pltpu.sync_copy(send_hbm.at[pl.ds(off, CH)], sidx_v)
            pltpu.sync_copy(recv_hbm.at[pl.ds(off, CH)], ridx_v)
            pltpu.sync_copy(coeff_hbm.at[pl.ds(off, CH), :], coef_v)
            pltpu.async_copy(x_hbm.at[sidx_v], rows_v, sem).wait()

            def row(j, c2):
                for k in range(d // L):
                    sl = pl.ds(k * L, L)
                    rows_v[j, sl] = rows_v[j, sl] * coef_v[j, sl]
                return c2
            lax.fori_loop(0, CH, row, 0)

            pltpu.sync_copy(rows_v, agg_sh.at[ridx_v], add=True)
            return carry

        lax.fori_loop(0, n_chunks, chunk, 0)
        plsc.subcore_barrier()

        pltpu.sync_copy(agg_sh.at[pl.ds(sid * rows_per_sid, rows_per_sid)],
                        out_hbm.at[cid, pl.ds(sid * rows_per_sid, rows_per_sid)])

    return sc_conv


def _out_body(p_ref, nf_ref, sp_ref, wout_ref, wsc_ref, o_ref, *, inv_sqrt_neigh, s):
    agg = (p_ref[0] + p_ref[1]) * inv_sqrt_neigh
    out = jnp.dot(agg, wout_ref[...], preferred_element_type=jnp.float32)
    nf = nf_ref[...]
    sp = sp_ref[...]
    sc = jnp.zeros_like(out)
    for k in range(s):
        cand = jnp.dot(nf, wsc_ref[k], preferred_element_type=jnp.float32)
        sc = jnp.where(sp == k, cand, sc)
    z = out + sc
    o_ref[...] = z * jax.nn.sigmoid(z)


def _finalize(partials, nf, species2d, W_out, W_sc, inv_sqrt_neigh, bn):
    n, d = nf.shape
    s = W_sc.shape[0]
    body = functools.partial(_out_body, inv_sqrt_neigh=inv_sqrt_neigh, s=s)
    return pl.pallas_call(
        body,
        grid=(n // bn,),
        in_specs=[
            pl.BlockSpec((NC, bn, d), lambda i: (0, i, 0)),
            pl.BlockSpec((bn, d), lambda i: (i, 0)),
            pl.BlockSpec((bn, 1), lambda i: (i, 0)),
            pl.BlockSpec((d, d), lambda i: (0, 0)),
            pl.BlockSpec((s, d, d), lambda i: (0, 0, 0)),
        ],
        out_specs=pl.BlockSpec((bn, d), lambda i: (i, 0)),
        out_shape=jax.ShapeDtypeStruct((n, d), jnp.float32),
    )(partials, nf, species2d, W_out, W_sc)


def kernel(node_features, edge_attributes, radial_embeddings, senders,
           receivers, species, W_in, W_r0, W_r1, W_attr, W_out, W_sc):
    n, d = node_features.shape
    e = senders.shape[0]
    avg_neigh = 32.0

    gran = NC * NS * CH
    ep = ((e + gran - 1) // gran) * gran
    pad = ep - e
    if pad:
        radial_p = jnp.pad(radial_embeddings, ((0, pad), (0, 0)))
        attrs_p = jnp.pad(edge_attributes, ((0, pad), (0, 0)))
        send_p = jnp.pad(senders.astype(jnp.int32), (0, pad))
        recv_p = jnp.pad(receivers.astype(jnp.int32), (0, pad))
    else:
        radial_p, attrs_p = radial_embeddings, edge_attributes
        send_p = senders.astype(jnp.int32)
        recv_p = receivers.astype(jnp.int32)

    coeff = _coeff(radial_p, attrs_p, W_r0, W_r1, W_attr, be=2048)
    x = _in_linear(node_features, W_in, bn=1000)

    ngran = NS * 8
    npad = ((n + ngran - 1) // ngran) * ngran
    zeros = jnp.zeros((npad, d), jnp.float32)
    sc_conv = _make_sc_conv(npad, d, ep)
    partials = sc_conv(x, coeff, send_p, recv_p, zeros)

    species2d = species.astype(jnp.int32).reshape(n, 1)
    return _finalize(partials, node_features, species2d, W_out, W_sc,
                     1.0 / (avg_neigh ** 0.5), bn=1000)


# speedup vs baseline: 2.2939x; 1.2820x over previous
import functools

import jax
import jax.numpy as jnp
from jax import lax
from jax.experimental import pallas as pl
from jax.experimental.pallas import tpu as pltpu
from jax.experimental.pallas import tpu_sc as plsc

NC = 2
NS = 16
L = 16
CH = 128


def _coeff_body(r_ref, a_ref, wr0_ref, wr1_ref, wattr_ref, o0_ref, o1_ref):
    r = r_ref[...]
    h = jax.nn.swish(jnp.dot(r, wr0_ref[...], preferred_element_type=jnp.float32))
    rad = jnp.dot(h, wr1_ref[...], preferred_element_type=jnp.float32)
    am = lax.dot_general(a_ref[...], wattr_ref[...], (((1,), (1,)), ((), ())),
                         preferred_element_type=jnp.float32)
    res = rad * am
    half = res.shape[1] // 2
    o0_ref[...] = res[:, :half]
    o1_ref[...] = res[:, half:]


def _coeff(radial, attrs, W_r0, W_r1, W_attr, be):
    e, r_dim = radial.shape
    de = attrs.shape[1]
    d = W_r1.shape[1]
    dh = d // 2
    return pl.pallas_call(
        _coeff_body,
        grid=(e // be,),
        in_specs=[
            pl.BlockSpec((be, r_dim), lambda i: (i, 0)),
            pl.BlockSpec((be, de), lambda i: (i, 0)),
            pl.BlockSpec((r_dim, r_dim), lambda i: (0, 0)),
            pl.BlockSpec((r_dim, d), lambda i: (0, 0)),
            pl.BlockSpec((d, de), lambda i: (0, 0)),
        ],
        out_specs=[pl.BlockSpec((be, dh), lambda i: (i, 0)),
                   pl.BlockSpec((be, dh), lambda i: (i, 0))],
        out_shape=[jax.ShapeDtypeStruct((e, dh), jnp.float32),
                   jax.ShapeDtypeStruct((e, dh), jnp.float32)],
    )(radial, attrs, W_r0, W_r1, W_attr)


def _matmul_body(x_ref, w_ref, o0_ref, o1_ref):
    res = jnp.dot(x_ref[...], w_ref[...], preferred_element_type=jnp.float32)
    half = res.shape[1] // 2
    o0_ref[...] = res[:, :half]
    o1_ref[...] = res[:, half:]


def _in_linear(nf, W_in, bn):
    n, d = nf.shape
    dh = d // 2
    return pl.pallas_call(
        _matmul_body,
        grid=(n // bn,),
        in_specs=[
            pl.BlockSpec((bn, d), lambda i: (i, 0)),
            pl.BlockSpec((d, d), lambda i: (0, 0)),
        ],
        out_specs=[pl.BlockSpec((bn, dh), lambda i: (i, 0)),
                   pl.BlockSpec((bn, dh), lambda i: (i, 0))],
        out_shape=[jax.ShapeDtypeStruct((n, dh), jnp.float32),
                   jax.ShapeDtypeStruct((n, dh), jnp.float32)],
    )(nf, W_in)


def _make_sc_conv(n, dh, ep, e_real):
    nw = NC * NS
    per_tile = ep // nw
    n_chunks = per_tile // CH
    rows_per_sid = n // NS
    zblocks = [(o, min(CH, rows_per_sid - o)) for o in range(0, rows_per_sid, CH)]
    mesh = plsc.VectorSubcoreMesh(core_axis_name="c", subcore_axis_name="s")

    @functools.partial(
        pl.kernel,
        out_type=jax.ShapeDtypeStruct((NC, n, dh), jnp.float32),
        mesh=mesh,
        compiler_params=pltpu.CompilerParams(use_tc_tiling_on_sc=False),
        scratch_types=[
            pltpu.VMEM((n_chunks, CH), jnp.int32),
            pltpu.VMEM((n_chunks, CH), jnp.int32),
            pltpu.VMEM((CH, dh), jnp.float32),
            pltpu.VMEM((CH, dh), jnp.float32),
            pltpu.VMEM((CH, dh), jnp.float32),
            pltpu.VMEM((CH, dh), jnp.float32),
            pltpu.VMEM_SHARED((n, dh), jnp.float32),
            pltpu.SemaphoreType.DMA,
            pltpu.SemaphoreType.DMA,
            pltpu.SemaphoreType.DMA,
            pltpu.SemaphoreType.DMA,
        ],
    )
    def sc_conv(x_hbm, coeff_hbm, send_hbm, recv_hbm, out_hbm,
                sidx_v, ridx_v, rows0, rows1, coef0, coef1, agg_sh,
                gs0, gs1, cs0, cs1):
        cid = lax.axis_index("c")
        sid = lax.axis_index("s")
        wid = cid * NS + sid
        base = wid * per_tile
        nrc = jnp.clip((e_real - base + CH - 1) // CH, 0, n_chunks)

        def zrow(j, c2):
            for k in range(dh // L):
                coef0[j, pl.ds(k * L, L)] = jnp.zeros((L,), jnp.float32)
            return c2
        lax.fori_loop(0, CH, zrow, 0)
        for zo, zs in zblocks:
            pltpu.sync_copy(coef0.at[pl.ds(0, zs)],
                            agg_sh.at[pl.ds(sid * rows_per_sid + zo, zs)])
        plsc.subcore_barrier()

        pltpu.sync_copy(send_hbm.at[wid], sidx_v)
        pltpu.sync_copy(recv_hbm.at[wid], ridx_v)

        def issue(t, rows, coef, gsem, csem):
            off = base + t * CH
            pltpu.async_copy(coeff_hbm.at[pl.ds(off, CH), :], coef, csem)
            pltpu.async_copy(x_hbm.at[sidx_v.at[t]], rows, gsem)

        def process(t, rows, coef, gsem, csem):
            pltpu.make_async_copy(coeff_hbm.at[pl.ds(0, CH), :], coef, csem).wait()
            pltpu.make_async_copy(x_hbm.at[sidx_v.at[0]], rows, gsem).wait()

            def row(j, c2):
                for k in range(dh // L):
                    sl = pl.ds(k * L, L)
                    rows[j, sl] = rows[j, sl] * coef[j, sl]
                return c2
            lax.fori_loop(0, CH, row, 0)
            pltpu.sync_copy(rows, agg_sh.at[ridx_v.at[t]], add=True)

        @pl.when(nrc > 0)
        def _():
            issue(0, rows0, coef0, gs0, cs0)

        def pair(u, carry):
            c = 2 * u

            @pl.when(c + 1 < nrc)
            def _():
                issue(c + 1, rows1, coef1, gs1, cs1)

            @pl.when(c < nrc)
            def _():
                process(c, rows0, coef0, gs0, cs0)

            @pl.when(c + 2 < nrc)
            def _():
                issue(c + 2, rows0, coef0, gs0, cs0)

            @pl.when(c + 1 < nrc)
            def _():
                process(c + 1, rows1, coef1, gs1, cs1)
            return carry

        lax.fori_loop(0, (n_chunks + 1) // 2, pair, 0)
        plsc.subcore_barrier()

        pltpu.sync_copy(agg_sh.at[pl.ds(sid * rows_per_sid, rows_per_sid)],
                        out_hbm.at[cid, pl.ds(sid * rows_per_sid, rows_per_sid)])

    return sc_conv


def _out_body(p0_ref, p1_ref, nf_ref, sp_ref, wout_ref, wsc_ref, o_ref, *,
              inv_sqrt_neigh, s, dh):
    agg0 = (p0_ref[0] + p0_ref[1]) * inv_sqrt_neigh
    agg1 = (p1_ref[0] + p1_ref[1]) * inv_sqrt_neigh
    out = (jnp.dot(agg0, wout_ref[pl.ds(0, dh), :], preferred_element_type=jnp.float32)
           + jnp.dot(agg1, wout_ref[pl.ds(dh, dh), :], preferred_element_type=jnp.float32))
    nf = nf_ref[...]
    sp = sp_ref[...]
    sc = jnp.zeros_like(out)
    for k in range(s):
        cand = jnp.dot(nf, wsc_ref[k], preferred_element_type=jnp.float32)
        sc = jnp.where(sp == k, cand, sc)
    z = out + sc
    o_ref[...] = z * jax.nn.sigmoid(z)


def _finalize(p0, p1, nf, species2d, W_out, W_sc, inv_sqrt_neigh, bn):
    n, d = nf.shape
    dh = d // 2
    s = W_sc.shape[0]
    body = functools.partial(_out_body, inv_sqrt_neigh=inv_sqrt_neigh, s=s, dh=dh)
    return pl.pallas_call(
        body,
        grid=(n // bn,),
        in_specs=[
            pl.BlockSpec((NC, bn, dh), lambda i: (0, i, 0)),
            pl.BlockSpec((NC, bn, dh), lambda i: (0, i, 0)),
            pl.BlockSpec((bn, d), lambda i: (i, 0)),
            pl.BlockSpec((bn, 1), lambda i: (i, 0)),
            pl.BlockSpec((d, d), lambda i: (0, 0)),
            pl.BlockSpec((s, d, d), lambda i: (0, 0, 0)),
        ],
        out_specs=pl.BlockSpec((bn, d), lambda i: (i, 0)),
        out_shape=jax.ShapeDtypeStruct((n, d), jnp.float32),
    )(p0, p1, nf, species2d, W_out, W_sc)


def kernel(node_features, edge_attributes, radial_embeddings, senders,
           receivers, species, W_in, W_r0, W_r1, W_attr, W_out, W_sc):
    n, d = node_features.shape
    e = senders.shape[0]
    avg_neigh = 32.0

    nw = NC * NS
    gran = nw * CH
    ep = ((e + gran - 1) // gran) * gran
    pad = ep - e
    send_p = jnp.pad(senders.astype(jnp.int32), (0, pad)).reshape(nw, ep // gran, CH)
    recv_p = jnp.pad(receivers.astype(jnp.int32), (0, pad)).reshape(nw, ep // gran, CH)

    c0, c1 = _coeff(radial_embeddings, edge_attributes, W_r0, W_r1, W_attr, be=2000)
    x0, x1 = _in_linear(node_features, W_in, bn=1000)

    ngran = NS * 8
    npad = ((n + ngran - 1) // ngran) * ngran
    sc_conv = _make_sc_conv(npad, d // 2, ep, e)
    p0 = sc_conv(x0, c0, send_p, recv_p)
    p1 = sc_conv(x1, c1, send_p, recv_p)

    species2d = species.astype(jnp.int32).reshape(n, 1)
    return _finalize(p0, p1, node_features, species2d, W_out, W_sc,
                     1.0 / (avg_neigh ** 0.5), bn=1000)


# speedup vs baseline: 2.9072x; 1.2674x over previous
import functools

import jax
import jax.numpy as jnp
from jax import lax
from jax.experimental import pallas as pl
from jax.experimental.pallas import tpu as pltpu
from jax.experimental.pallas import tpu_sc as plsc

NC = 2
NS = 16
L = 16
CH = 128


def _coeff_body(r_ref, a_ref, wr0b_ref, w1b0_ref, w1b1_ref, wab0_ref, wab1_ref,
                o0_ref, o1_ref):
    h = jax.nn.swish(jnp.dot(r_ref[...], wr0b_ref[...],
                             preferred_element_type=jnp.float32))
    a = a_ref[...]
    o0_ref[...] = (jnp.dot(h, w1b0_ref[...], preferred_element_type=jnp.float32)
                   * jnp.dot(a, wab0_ref[...], preferred_element_type=jnp.float32))
    o1_ref[...] = (jnp.dot(h, w1b1_ref[...], preferred_element_type=jnp.float32)
                   * jnp.dot(a, wab1_ref[...], preferred_element_type=jnp.float32))


def _blockdiag(w):
    r, c = w.shape
    z = jnp.zeros((r, c), w.dtype)
    return jnp.concatenate([jnp.concatenate([w, z], 1),
                            jnp.concatenate([z, w], 1)], 0)


def _coeff(radial, attrs, W_r0, W_r1, W_attr, be2):
    e, r_dim = radial.shape
    de = attrs.shape[1]
    d = W_r1.shape[1]
    dh = d // 2
    e2 = e // 2
    r2 = radial.reshape(e2, 2 * r_dim)
    a2 = attrs.reshape(e2, 2 * de)
    wr0b = _blockdiag(W_r0)
    w1b0 = _blockdiag(W_r1[:, :dh])
    w1b1 = _blockdiag(W_r1[:, dh:])
    wab0 = _blockdiag(W_attr[:dh, :].T)
    wab1 = _blockdiag(W_attr[dh:, :].T)
    return pl.pallas_call(
        _coeff_body,
        grid=(e2 // be2,),
        in_specs=[
            pl.BlockSpec((be2, 2 * r_dim), lambda i: (i, 0)),
            pl.BlockSpec((be2, 2 * de), lambda i: (i, 0)),
            pl.BlockSpec((2 * r_dim, 2 * r_dim), lambda i: (0, 0)),
            pl.BlockSpec((2 * r_dim, d), lambda i: (0, 0)),
            pl.BlockSpec((2 * r_dim, d), lambda i: (0, 0)),
            pl.BlockSpec((2 * de, d), lambda i: (0, 0)),
            pl.BlockSpec((2 * de, d), lambda i: (0, 0)),
        ],
        out_specs=[pl.BlockSpec((be2, d), lambda i: (i, 0)),
                   pl.BlockSpec((be2, d), lambda i: (i, 0))],
        out_shape=[jax.ShapeDtypeStruct((e2, d), jnp.float32),
                   jax.ShapeDtypeStruct((e2, d), jnp.float32)],
    )(r2, a2, wr0b, w1b0, w1b1, wab0, wab1)


def _matmul_body(x_ref, w_ref, o0_ref, o1_ref):
    res = jnp.dot(x_ref[...], w_ref[...], preferred_element_type=jnp.float32)
    half = res.shape[1] // 2
    o0_ref[...] = res[:, :half]
    o1_ref[...] = res[:, half:]


def _in_linear(nf, W_in, bn):
    n, d = nf.shape
    dh = d // 2
    return pl.pallas_call(
        _matmul_body,
        grid=(n // bn,),
        in_specs=[
            pl.BlockSpec((bn, d), lambda i: (i, 0)),
            pl.BlockSpec((d, d), lambda i: (0, 0)),
        ],
        out_specs=[pl.BlockSpec((bn, dh), lambda i: (i, 0)),
                   pl.BlockSpec((bn, dh), lambda i: (i, 0))],
        out_shape=[jax.ShapeDtypeStruct((n, dh), jnp.float32),
                   jax.ShapeDtypeStruct((n, dh), jnp.float32)],
    )(nf, W_in)


def _make_sc_conv(n, dh, ep, e_real):
    nw = NC * NS
    per_tile = ep // nw
    n_chunks = per_tile // CH
    rows_per_sid = n // NS
    zblocks = [(o, min(CH, rows_per_sid - o)) for o in range(0, rows_per_sid, CH)]
    mesh = plsc.VectorSubcoreMesh(core_axis_name="c", subcore_axis_name="s")

    @functools.partial(
        pl.kernel,
        out_type=jax.ShapeDtypeStruct((NC, n, dh), jnp.float32),
        mesh=mesh,
        compiler_params=pltpu.CompilerParams(use_tc_tiling_on_sc=False),
        scratch_types=[
            pltpu.VMEM((n_chunks, CH), jnp.int32),
            pltpu.VMEM((n_chunks, CH), jnp.int32),
            pltpu.VMEM((CH, dh), jnp.float32),
            pltpu.VMEM((CH, dh), jnp.float32),
            pltpu.VMEM((CH // 2, 2 * dh), jnp.float32),
            pltpu.VMEM((CH // 2, 2 * dh), jnp.float32),
            pltpu.VMEM_SHARED((n, dh), jnp.float32),
            pltpu.SemaphoreType.DMA,
            pltpu.SemaphoreType.DMA,
            pltpu.SemaphoreType.DMA,
            pltpu.SemaphoreType.DMA,
        ],
    )
    def sc_conv(x_hbm, coeff_hbm, send_hbm, recv_hbm, out_hbm,
                sidx_v, ridx_v, rows0, rows1, coef0, coef1, agg_sh,
                gs0, gs1, cs0, cs1):
        cid = lax.axis_index("c")
        sid = lax.axis_index("s")
        wid = cid * NS + sid
        base = wid * per_tile
        nrc = jnp.clip((e_real - base + CH - 1) // CH, 0, n_chunks)

        def zrow(j, c2):
            for k in range(dh // L):
                rows0[j, pl.ds(k * L, L)] = jnp.zeros((L,), jnp.float32)
            return c2
        lax.fori_loop(0, CH, zrow, 0)
        for zo, zs in zblocks:
            pltpu.sync_copy(rows0.at[pl.ds(0, zs)],
                            agg_sh.at[pl.ds(sid * rows_per_sid + zo, zs)])
        plsc.subcore_barrier()

        pltpu.sync_copy(send_hbm.at[pl.ds(wid * n_chunks, n_chunks), :], sidx_v)
        pltpu.sync_copy(recv_hbm.at[pl.ds(wid * n_chunks, n_chunks), :], ridx_v)

        def issue(t, rows, coef, gsem, csem):
            off = base + t * CH
            pltpu.async_copy(coeff_hbm.at[pl.ds(off // 2, CH // 2), :], coef, csem)
            pltpu.async_copy(x_hbm.at[sidx_v.at[t]], rows, gsem)

        def process(t, rows, coef, gsem, csem):
            pltpu.make_async_copy(coeff_hbm.at[pl.ds(0, CH // 2), :], coef, csem).wait()
            pltpu.make_async_copy(x_hbm.at[sidx_v.at[0]], rows, gsem).wait()

            def pairrow(p, c2):
                for half in range(2):
                    for k in range(dh // L):
                        rows[2 * p + half, pl.ds(k * L, L)] = (
                            rows[2 * p + half, pl.ds(k * L, L)]
                            * coef[p, pl.ds(half * dh + k * L, L)])
                return c2
            lax.fori_loop(0, CH // 2, pairrow, 0)
            pltpu.sync_copy(rows, agg_sh.at[ridx_v.at[t]], add=True)

        @pl.when(nrc > 0)
        def _():
            issue(0, rows0, coef0, gs0, cs0)

        def pair(u, carry):
            c = 2 * u

            @pl.when(c + 1 < nrc)
            def _():
                issue(c + 1, rows1, coef1, gs1, cs1)

            @pl.when(c < nrc)
            def _():
                process(c, rows0, coef0, gs0, cs0)

            @pl.when(c + 2 < nrc)
            def _():
                issue(c + 2, rows0, coef0, gs0, cs0)

            @pl.when(c + 1 < nrc)
            def _():
                process(c + 1, rows1, coef1, gs1, cs1)
            return carry

        lax.fori_loop(0, (n_chunks + 1) // 2, pair, 0)
        plsc.subcore_barrier()

        pltpu.sync_copy(agg_sh.at[pl.ds(sid * rows_per_sid, rows_per_sid)],
                        out_hbm.at[cid, pl.ds(sid * rows_per_sid, rows_per_sid)])

    return sc_conv


def _out_body(p0_ref, p1_ref, nf_ref, sp_ref, wout_ref, wsc_ref, o_ref, *,
              inv_sqrt_neigh, s, dh):
    agg0 = (p0_ref[0] + p0_ref[1]) * inv_sqrt_neigh
    agg1 = (p1_ref[0] + p1_ref[1]) * inv_sqrt_neigh
    out = (jnp.dot(agg0, wout_ref[pl.ds(0, dh), :], preferred_element_type=jnp.float32)
           + jnp.dot(agg1, wout_ref[pl.ds(dh, dh), :], preferred_element_type=jnp.float32))
    nf = nf_ref[...]
    sp = sp_ref[...]
    sc = jnp.zeros_like(out)
    for k in range(s):
        cand = jnp.dot(nf, wsc_ref[k], preferred_element_type=jnp.float32)
        sc = jnp.where(sp == k, cand, sc)
    z = out + sc
    o_ref[...] = z * jax.nn.sigmoid(z)


def _finalize(p0, p1, nf, species2d, W_out, W_sc, inv_sqrt_neigh, bn):
    n, d = nf.shape
    dh = d // 2
    s = W_sc.shape[0]
    body = functools.partial(_out_body, inv_sqrt_neigh=inv_sqrt_neigh, s=s, dh=dh)
    return pl.pallas_call(
        body,
        grid=(n // bn,),
        in_specs=[
            pl.BlockSpec((NC, bn, dh), lambda i: (0, i, 0)),
            pl.BlockSpec((NC, bn, dh), lambda i: (0, i, 0)),
            pl.BlockSpec((bn, d), lambda i: (i, 0)),
            pl.BlockSpec((bn, 1), lambda i: (i, 0)),
            pl.BlockSpec((d, d), lambda i: (0, 0)),
            pl.BlockSpec((s, d, d), lambda i: (0, 0, 0)),
        ],
        out_specs=pl.BlockSpec((bn, d), lambda i: (i, 0)),
        out_shape=jax.ShapeDtypeStruct((n, d), jnp.float32),
    )(p0, p1, nf, species2d, W_out, W_sc)


def kernel(node_features, edge_attributes, radial_embeddings, senders,
           receivers, species, W_in, W_r0, W_r1, W_attr, W_out, W_sc):
    n, d = node_features.shape
    e = senders.shape[0]
    avg_neigh = 32.0

    nw = NC * NS
    gran = nw * CH
    ep = ((e + gran - 1) // gran) * gran
    pad = ep - e
    send_p = jnp.pad(senders.astype(jnp.int32), (0, pad)).reshape(ep // CH, CH)
    recv_p = jnp.pad(receivers.astype(jnp.int32), (0, pad)).reshape(ep // CH, CH)

    c0, c1 = _coeff(radial_embeddings, edge_attributes, W_r0, W_r1, W_attr, be2=2000)
    x0, x1 = _in_linear(node_features, W_in, bn=1000)

    ngran = NS * 8
    npad = ((n + ngran - 1) // ngran) * ngran
    sc_conv = _make_sc_conv(npad, d // 2, ep, e)
    p0 = sc_conv(x0, c0, send_p, recv_p)
    p1 = sc_conv(x1, c1, send_p, recv_p)

    species2d = species.astype(jnp.int32).reshape(n, 1)
    return _finalize(p0, p1, node_features, species2d, W_out, W_sc,
                     1.0 / (avg_neigh ** 0.5), bn=1000)


# speedup vs baseline: 3.0189x; 1.0384x over previous
import functools

import jax
import jax.numpy as jnp
from jax import lax
from jax.experimental import pallas as pl
from jax.experimental.pallas import tpu as pltpu
from jax.experimental.pallas import tpu_sc as plsc

NC = 2
NS = 16
L = 16
CH = 128


def _coeff_body(r_ref, a_ref, wr0_ref, wr1_ref, wattr_ref, o_ref):
    r = r_ref[...]
    h = jax.nn.swish(jnp.dot(r, wr0_ref[...], preferred_element_type=jnp.float32))
    rad = jnp.dot(h, wr1_ref[...], preferred_element_type=jnp.float32)
    am = lax.dot_general(a_ref[...], wattr_ref[...], (((1,), (1,)), ((), ())),
                         preferred_element_type=jnp.float32)
    o_ref[...] = rad * am


def _coeff(radial, attrs, W_r0, W_r1, W_attr, be):
    e, r_dim = radial.shape
    de = attrs.shape[1]
    d = W_r1.shape[1]
    return pl.pallas_call(
        _coeff_body,
        grid=(e // be,),
        in_specs=[
            pl.BlockSpec((be, r_dim), lambda i: (i, 0)),
            pl.BlockSpec((be, de), lambda i: (i, 0)),
            pl.BlockSpec((r_dim, r_dim), lambda i: (0, 0)),
            pl.BlockSpec((r_dim, d), lambda i: (0, 0)),
            pl.BlockSpec((d, de), lambda i: (0, 0)),
        ],
        out_specs=pl.BlockSpec((be, d), lambda i: (i, 0)),
        out_shape=jax.ShapeDtypeStruct((e, d), jnp.float32),
    )(radial, attrs, W_r0, W_r1, W_attr)


def _matmul_body(x_ref, w_ref, o0_ref, o1_ref):
    res = jnp.dot(x_ref[...], w_ref[...], preferred_element_type=jnp.float32)
    half = res.shape[1] // 2
    o0_ref[...] = res[:, :half]
    o1_ref[...] = res[:, half:]


def _in_linear(nf, W_in, bn):
    n, d = nf.shape
    dh = d // 2
    return pl.pallas_call(
        _matmul_body,
        grid=(n // bn,),
        in_specs=[
            pl.BlockSpec((bn, d), lambda i: (i, 0)),
            pl.BlockSpec((d, d), lambda i: (0, 0)),
        ],
        out_specs=[pl.BlockSpec((bn, dh), lambda i: (i, 0)),
                   pl.BlockSpec((bn, dh), lambda i: (i, 0))],
        out_shape=[jax.ShapeDtypeStruct((n, dh), jnp.float32),
                   jax.ShapeDtypeStruct((n, dh), jnp.float32)],
    )(nf, W_in)


def _make_sc_conv(n, dh, ep, e_real, col):
    nw = NC * NS
    per_tile = ep // nw
    n_chunks = per_tile // CH
    rows_per_sid = n // NS
    zblocks = [(o, min(CH, rows_per_sid - o)) for o in range(0, rows_per_sid, CH)]
    mesh = plsc.VectorSubcoreMesh(core_axis_name="c", subcore_axis_name="s")

    @functools.partial(
        pl.kernel,
        out_type=jax.ShapeDtypeStruct((NC, n, dh), jnp.float32),
        mesh=mesh,
        compiler_params=pltpu.CompilerParams(use_tc_tiling_on_sc=False),
        scratch_types=[
            pltpu.VMEM((n_chunks, CH), jnp.int32),
            pltpu.VMEM((n_chunks, CH), jnp.int32),
            pltpu.VMEM((CH, dh), jnp.float32),
            pltpu.VMEM((CH, dh), jnp.float32),
            pltpu.VMEM((CH, dh), jnp.float32),
            pltpu.VMEM((CH, dh), jnp.float32),
            pltpu.VMEM_SHARED((n, dh), jnp.float32),
            pltpu.SemaphoreType.DMA,
            pltpu.SemaphoreType.DMA,
            pltpu.SemaphoreType.DMA,
            pltpu.SemaphoreType.DMA,
        ],
    )
    def sc_conv(x_hbm, coeff_hbm, send_hbm, recv_hbm, out_hbm,
                sidx_v, ridx_v, rows0, rows1, coef0, coef1, agg_sh,
                gs0, gs1, cs0, cs1):
        cid = lax.axis_index("c")
        sid = lax.axis_index("s")
        wid = cid * NS + sid
        base = wid * per_tile
        nrc = jnp.clip((e_real - base + CH - 1) // CH, 0, n_chunks)

        def zrow(j, c2):
            for k in range(dh // L):
                rows0[j, pl.ds(k * L, L)] = jnp.zeros((L,), jnp.float32)
            return c2
        lax.fori_loop(0, CH, zrow, 0)
        for zo, zs in zblocks:
            pltpu.sync_copy(rows0.at[pl.ds(0, zs)],
                            agg_sh.at[pl.ds(sid * rows_per_sid + zo, zs)])
        plsc.subcore_barrier()

        pltpu.sync_copy(send_hbm.at[pl.ds(wid * n_chunks, n_chunks), :], sidx_v)
        pltpu.sync_copy(recv_hbm.at[pl.ds(wid * n_chunks, n_chunks), :], ridx_v)

        def issue(t, rows, coef, gsem, csem):
            off = base + t * CH
            pltpu.async_copy(coeff_hbm.at[pl.ds(off, CH), pl.ds(col, dh)], coef, csem)
            pltpu.async_copy(x_hbm.at[sidx_v.at[t]], rows, gsem)

        def process(t, rows, coef, gsem, csem):
            pltpu.make_async_copy(coeff_hbm.at[pl.ds(0, CH), pl.ds(col, dh)], coef, csem).wait()
            pltpu.make_async_copy(x_hbm.at[sidx_v.at[0]], rows, gsem).wait()

            def row(j, c2):
                for k in range(dh // L):
                    sl = pl.ds(k * L, L)
                    rows[j, sl] = rows[j, sl] * coef[j, sl]
                return c2
            lax.fori_loop(0, CH, row, 0)
            pltpu.sync_copy(rows, agg_sh.at[ridx_v.at[t]], add=True)

        @pl.when(nrc > 0)
        def _():
            issue(0, rows0, coef0, gs0, cs0)

        def pair(u, carry):
            c = 2 * u

            @pl.when(c + 1 < nrc)
            def _():
                issue(c + 1, rows1, coef1, gs1, cs1)

            @pl.when(c < nrc)
            def _():
                process(c, rows0, coef0, gs0, cs0)

            @pl.when(c + 2 < nrc)
            def _():
                issue(c + 2, rows0, coef0, gs0, cs0)

            @pl.when(c + 1 < nrc)
            def _():
                process(c + 1, rows1, coef1, gs1, cs1)
            return carry

        lax.fori_loop(0, (n_chunks + 1) // 2, pair, 0)
        plsc.subcore_barrier()

        pltpu.sync_copy(agg_sh.at[pl.ds(sid * rows_per_sid, rows_per_sid)],
                        out_hbm.at[cid, pl.ds(sid * rows_per_sid, rows_per_sid)])

    return sc_conv


def _out_body(p0_ref, p1_ref, nf_ref, sp_ref, wout_ref, wsc_ref, o_ref, *,
              inv_sqrt_neigh, s, dh):
    agg0 = (p0_ref[0] + p0_ref[1]) * inv_sqrt_neigh
    agg1 = (p1_ref[0] + p1_ref[1]) * inv_sqrt_neigh
    out = (jnp.dot(agg0, wout_ref[pl.ds(0, dh), :], preferred_element_type=jnp.float32)
           + jnp.dot(agg1, wout_ref[pl.ds(dh, dh), :], preferred_element_type=jnp.float32))
    nf = nf_ref[...]
    sp = sp_ref[...]
    sc = jnp.zeros_like(out)
    for k in range(s):
        cand = jnp.dot(nf, wsc_ref[k], preferred_element_type=jnp.float32)
        sc = jnp.where(sp == k, cand, sc)
    z = out + sc
    o_ref[...] = z * jax.nn.sigmoid(z)


def _finalize(p0, p1, nf, species2d, W_out, W_sc, inv_sqrt_neigh, bn):
    n, d = nf.shape
    dh = d // 2
    s = W_sc.shape[0]
    body = functools.partial(_out_body, inv_sqrt_neigh=inv_sqrt_neigh, s=s, dh=dh)
    return pl.pallas_call(
        body,
        grid=(n // bn,),
        in_specs=[
            pl.BlockSpec((NC, bn, dh), lambda i: (0, i, 0)),
            pl.BlockSpec((NC, bn, dh), lambda i: (0, i, 0)),
            pl.BlockSpec((bn, d), lambda i: (i, 0)),
            pl.BlockSpec((bn, 1), lambda i: (i, 0)),
            pl.BlockSpec((d, d), lambda i: (0, 0)),
            pl.BlockSpec((s, d, d), lambda i: (0, 0, 0)),
        ],
        out_specs=pl.BlockSpec((bn, d), lambda i: (i, 0)),
        out_shape=jax.ShapeDtypeStruct((n, d), jnp.float32),
    )(p0, p1, nf, species2d, W_out, W_sc)


def kernel(node_features, edge_attributes, radial_embeddings, senders,
           receivers, species, W_in, W_r0, W_r1, W_attr, W_out, W_sc):
    n, d = node_features.shape
    e = senders.shape[0]
    avg_neigh = 32.0

    nw = NC * NS
    gran = nw * CH
    ep = ((e + gran - 1) // gran) * gran
    pad = ep - e
    send_p = jnp.pad(senders.astype(jnp.int32), (0, pad)).reshape(ep // CH, CH)
    recv_p = jnp.pad(receivers.astype(jnp.int32), (0, pad)).reshape(ep // CH, CH)

    coeff = _coeff(radial_embeddings, edge_attributes, W_r0, W_r1, W_attr, be=2000)
    x0, x1 = _in_linear(node_features, W_in, bn=1000)

    ngran = NS * 8
    npad = ((n + ngran - 1) // ngran) * ngran
    p0 = _make_sc_conv(npad, d // 2, ep, e, 0)(x0, coeff, send_p, recv_p)
    p1 = _make_sc_conv(npad, d // 2, ep, e, d // 2)(x1, coeff, send_p, recv_p)

    species2d = species.astype(jnp.int32).reshape(n, 1)
    return _finalize(p0, p1, node_features, species2d, W_out, W_sc,
                     1.0 / (avg_neigh ** 0.5), bn=1000)


# speedup vs baseline: 4.7615x; 1.5773x over previous
import functools

import jax
import jax.numpy as jnp
from jax import lax
from jax.experimental import pallas as pl
from jax.experimental.pallas import tpu as pltpu
from jax.experimental.pallas import tpu_sc as plsc

NC = 2
NS = 16
L = 16
CH = 128


def _coeff_body(rt_ref, at_ref, wr0_ref, wr1_ref, wattr_ref, o_ref):
    h = jax.nn.swish(lax.dot_general(rt_ref[...], wr0_ref[...],
                                     (((0,), (0,)), ((), ())),
                                     preferred_element_type=jnp.float32))
    rad = jnp.dot(h, wr1_ref[...], preferred_element_type=jnp.float32)
    am = lax.dot_general(at_ref[...], wattr_ref[...], (((0,), (1,)), ((), ())),
                         preferred_element_type=jnp.float32)
    o_ref[...] = rad * am


def _coeff(radial, attrs, W_r0, W_r1, W_attr, be):
    e, r_dim = radial.shape
    de = attrs.shape[1]
    d = W_r1.shape[1]
    rt = radial.T
    at = attrs.T
    return pl.pallas_call(
        _coeff_body,
        grid=(e // be,),
        in_specs=[
            pl.BlockSpec((r_dim, be), lambda i: (0, i)),
            pl.BlockSpec((de, be), lambda i: (0, i)),
            pl.BlockSpec((r_dim, r_dim), lambda i: (0, 0)),
            pl.BlockSpec((r_dim, d), lambda i: (0, 0)),
            pl.BlockSpec((d, de), lambda i: (0, 0)),
        ],
        out_specs=pl.BlockSpec((be, d), lambda i: (i, 0)),
        out_shape=jax.ShapeDtypeStruct((e, d), jnp.float32),
    )(rt, at, W_r0, W_r1, W_attr)


def _matmul_body(x_ref, w_ref, o0_ref, o1_ref):
    res = jnp.dot(x_ref[...], w_ref[...], preferred_element_type=jnp.float32)
    half = res.shape[1] // 2
    o0_ref[...] = res[:, :half]
    o1_ref[...] = res[:, half:]


def _in_linear(nf, W_in, bn):
    n, d = nf.shape
    dh = d // 2
    return pl.pallas_call(
        _matmul_body,
        grid=(n // bn,),
        in_specs=[
            pl.BlockSpec((bn, d), lambda i: (i, 0)),
            pl.BlockSpec((d, d), lambda i: (0, 0)),
        ],
        out_specs=[pl.BlockSpec((bn, dh), lambda i: (i, 0)),
                   pl.BlockSpec((bn, dh), lambda i: (i, 0))],
        out_shape=[jax.ShapeDtypeStruct((n, dh), jnp.float32),
                   jax.ShapeDtypeStruct((n, dh), jnp.float32)],
    )(nf, W_in)


def _make_sc_conv(n, dh, ep, e_real, col):
    nw = NC * NS
    per_tile = ep // nw
    n_chunks = per_tile // CH
    rows_per_sid = n // NS
    zblocks = [(o, min(CH, rows_per_sid - o)) for o in range(0, rows_per_sid, CH)]
    mesh = plsc.VectorSubcoreMesh(core_axis_name="c", subcore_axis_name="s")

    @functools.partial(
        pl.kernel,
        out_type=jax.ShapeDtypeStruct((NC, n, dh), jnp.float32),
        mesh=mesh,
        compiler_params=pltpu.CompilerParams(use_tc_tiling_on_sc=False),
        scratch_types=[
            pltpu.VMEM((n_chunks, CH), jnp.int32),
            pltpu.VMEM((n_chunks, CH), jnp.int32),
            pltpu.VMEM((CH, dh), jnp.float32),
            pltpu.VMEM((CH, dh), jnp.float32),
            pltpu.VMEM((CH, dh), jnp.float32),
            pltpu.VMEM((CH, dh), jnp.float32),
            pltpu.VMEM_SHARED((n, dh), jnp.float32),
            pltpu.SemaphoreType.DMA,
            pltpu.SemaphoreType.DMA,
            pltpu.SemaphoreType.DMA,
            pltpu.SemaphoreType.DMA,
        ],
    )
    def sc_conv(x_hbm, coeff_hbm, send_hbm, recv_hbm, out_hbm,
                sidx_v, ridx_v, rows0, rows1, coef0, coef1, agg_sh,
                gs0, gs1, cs0, cs1):
        cid = lax.axis_index("c")
        sid = lax.axis_index("s")
        wid = cid * NS + sid
        base = wid * per_tile
        nrc = jnp.clip((e_real - base + CH - 1) // CH, 0, n_chunks)

        def zrow(j, c2):
            for k in range(dh // L):
                rows0[j, pl.ds(k * L, L)] = jnp.zeros((L,), jnp.float32)
            return c2
        lax.fori_loop(0, CH, zrow, 0)
        for zo, zs in zblocks:
            pltpu.sync_copy(rows0.at[pl.ds(0, zs)],
                            agg_sh.at[pl.ds(sid * rows_per_sid + zo, zs)])
        plsc.subcore_barrier()

        pltpu.sync_copy(send_hbm.at[pl.ds(wid * n_chunks, n_chunks), :], sidx_v)
        pltpu.sync_copy(recv_hbm.at[pl.ds(wid * n_chunks, n_chunks), :], ridx_v)

        def issue(t, rows, coef, gsem, csem):
            off = base + t * CH
            pltpu.async_copy(coeff_hbm.at[pl.ds(off, CH), pl.ds(col, dh)], coef, csem)
            pltpu.async_copy(x_hbm.at[sidx_v.at[t]], rows, gsem)

        def process(t, rows, coef, gsem, csem):
            pltpu.make_async_copy(coeff_hbm.at[pl.ds(0, CH), pl.ds(col, dh)], coef, csem).wait()
            pltpu.make_async_copy(x_hbm.at[sidx_v.at[0]], rows, gsem).wait()

            def row(j, c2):
                for k in range(dh // L):
                    sl = pl.ds(k * L, L)
                    rows[j, sl] = rows[j, sl] * coef[j, sl]
                return c2
            lax.fori_loop(0, CH, row, 0)
            pltpu.sync_copy(rows, agg_sh.at[ridx_v.at[t]], add=True)

        @pl.when(nrc > 0)
        def _():
            issue(0, rows0, coef0, gs0, cs0)

        def pair(u, carry):
            c = 2 * u

            @pl.when(c + 1 < nrc)
            def _():
                issue(c + 1, rows1, coef1, gs1, cs1)

            @pl.when(c < nrc)
            def _():
                process(c, rows0, coef0, gs0, cs0)

            @pl.when(c + 2 < nrc)
            def _():
                issue(c + 2, rows0, coef0, gs0, cs0)

            @pl.when(c + 1 < nrc)
            def _():
                process(c + 1, rows1, coef1, gs1, cs1)
            return carry

        lax.fori_loop(0, (n_chunks + 1) // 2, pair, 0)
        plsc.subcore_barrier()

        pltpu.sync_copy(agg_sh.at[pl.ds(sid * rows_per_sid, rows_per_sid)],
                        out_hbm.at[cid, pl.ds(sid * rows_per_sid, rows_per_sid)])

    return sc_conv


def _out_body(p0_ref, p1_ref, nf_ref, sp_ref, wout_ref, wsc_ref, o_ref, *,
              inv_sqrt_neigh, s, dh):
    agg0 = (p0_ref[0] + p0_ref[1]) * inv_sqrt_neigh
    agg1 = (p1_ref[0] + p1_ref[1]) * inv_sqrt_neigh
    out = (jnp.dot(agg0, wout_ref[pl.ds(0, dh), :], preferred_element_type=jnp.float32)
           + jnp.dot(agg1, wout_ref[pl.ds(dh, dh), :], preferred_element_type=jnp.float32))
    nf = nf_ref[...]
    sp = sp_ref[...]
    sc = jnp.zeros_like(out)
    for k in range(s):
        cand = jnp.dot(nf, wsc_ref[k], preferred_element_type=jnp.float32)
        sc = jnp.where(sp == k, cand, sc)
    z = out + sc
    o_ref[...] = z * jax.nn.sigmoid(z)


def _finalize(p0, p1, nf, species2d, W_out, W_sc, inv_sqrt_neigh, bn):
    n, d = nf.shape
    dh = d // 2
    s = W_sc.shape[0]
    body = functools.partial(_out_body, inv_sqrt_neigh=inv_sqrt_neigh, s=s, dh=dh)
    return pl.pallas_call(
        body,
        grid=(n // bn,),
        in_specs=[
            pl.BlockSpec((NC, bn, dh), lambda i: (0, i, 0)),
            pl.BlockSpec((NC, bn, dh), lambda i: (0, i, 0)),
            pl.BlockSpec((bn, d), lambda i: (i, 0)),
            pl.BlockSpec((bn, 1), lambda i: (i, 0)),
            pl.BlockSpec((d, d), lambda i: (0, 0)),
            pl.BlockSpec((s, d, d), lambda i: (0, 0, 0)),
        ],
        out_specs=pl.BlockSpec((bn, d), lambda i: (i, 0)),
        out_shape=jax.ShapeDtypeStruct((n, d), jnp.float32),
    )(p0, p1, nf, species2d, W_out, W_sc)


def kernel(node_features, edge_attributes, radial_embeddings, senders,
           receivers, species, W_in, W_r0, W_r1, W_attr, W_out, W_sc):
    n, d = node_features.shape
    e = senders.shape[0]
    avg_neigh = 32.0

    nw = NC * NS
    gran = nw * CH
    ep = ((e + gran - 1) // gran) * gran
    pad = ep - e
    send_p = jnp.pad(senders.astype(jnp.int32), (0, pad)).reshape(ep // CH, CH)
    recv_p = jnp.pad(receivers.astype(jnp.int32), (0, pad)).reshape(ep // CH, CH)

    coeff = _coeff(radial_embeddings, edge_attributes, W_r0, W_r1, W_attr, be=2560)
    x0, x1 = _in_linear(node_features, W_in, bn=1000)

    ngran = NS * 8
    npad = ((n + ngran - 1) // ngran) * ngran
    p0 = _make_sc_conv(npad, d // 2, ep, e, 0)(x0, coeff, send_p, recv_p)
    p1 = _make_sc_conv(npad, d // 2, ep, e, d // 2)(x1, coeff, send_p, recv_p)

    species2d = species.astype(jnp.int32).reshape(n, 1)
    return _finalize(p0, p1, node_features, species2d, W_out, W_sc,
                     1.0 / (avg_neigh ** 0.5), bn=1000)


# speedup vs baseline: 4.8698x; 1.0227x over previous
import functools

import jax
import jax.numpy as jnp
from jax import lax
from jax.experimental import pallas as pl
from jax.experimental.pallas import tpu as pltpu
from jax.experimental.pallas import tpu_sc as plsc

NC = 2
NS = 16
L = 16
CH = 128


def _coeff_body(rt_ref, at_ref, wr0t_ref, wr1t_ref, wattr_ref, o_ref, *, cb):
    ht = jax.nn.swish(jnp.dot(wr0t_ref[...], rt_ref[...],
                              preferred_element_type=jnp.float32))
    be = rt_ref.shape[1]
    wattr = wattr_ref[...]
    wr1t = wr1t_ref[...]
    for k in range(be // cb):
        lo, hi = k * cb, (k + 1) * cb
        radt = jnp.dot(wr1t, ht[:, lo:hi], preferred_element_type=jnp.float32)
        amt = jnp.dot(wattr, at_ref[:, lo:hi], preferred_element_type=jnp.float32)
        o_ref[lo:hi, :] = (radt * amt).T


def _coeff(radial, attrs, W_r0, W_r1, W_attr, be):
    e, r_dim = radial.shape
    de = attrs.shape[1]
    d = W_r1.shape[1]
    rt = radial.T
    at = attrs.T
    return pl.pallas_call(
        functools.partial(_coeff_body, cb=256),
        grid=(e // be,),
        in_specs=[
            pl.BlockSpec((r_dim, be), lambda i: (0, i)),
            pl.BlockSpec((de, be), lambda i: (0, i)),
            pl.BlockSpec((r_dim, r_dim), lambda i: (0, 0)),
            pl.BlockSpec((d, r_dim), lambda i: (0, 0)),
            pl.BlockSpec((d, de), lambda i: (0, 0)),
        ],
        out_specs=pl.BlockSpec((be, d), lambda i: (i, 0)),
        out_shape=jax.ShapeDtypeStruct((e, d), jnp.float32),
    )(rt, at, W_r0.T, W_r1.T, W_attr)


def _matmul_body(x_ref, w_ref, o0_ref, o1_ref):
    res = jnp.dot(x_ref[...], w_ref[...], preferred_element_type=jnp.float32)
    half = res.shape[1] // 2
    o0_ref[...] = res[:, :half]
    o1_ref[...] = res[:, half:]


def _in_linear(nf, W_in, bn):
    n, d = nf.shape
    dh = d // 2
    return pl.pallas_call(
        _matmul_body,
        grid=(n // bn,),
        in_specs=[
            pl.BlockSpec((bn, d), lambda i: (i, 0)),
            pl.BlockSpec((d, d), lambda i: (0, 0)),
        ],
        out_specs=[pl.BlockSpec((bn, dh), lambda i: (i, 0)),
                   pl.BlockSpec((bn, dh), lambda i: (i, 0))],
        out_shape=[jax.ShapeDtypeStruct((n, dh), jnp.float32),
                   jax.ShapeDtypeStruct((n, dh), jnp.float32)],
    )(nf, W_in)


def _make_sc_conv(n, dh, ep, e_real, col):
    nw = NC * NS
    per_tile = ep // nw
    n_chunks = per_tile // CH
    rows_per_sid = n // NS
    zblocks = [(o, min(CH, rows_per_sid - o)) for o in range(0, rows_per_sid, CH)]
    mesh = plsc.VectorSubcoreMesh(core_axis_name="c", subcore_axis_name="s")

    @functools.partial(
        pl.kernel,
        out_type=jax.ShapeDtypeStruct((NC, n, dh), jnp.float32),
        mesh=mesh,
        compiler_params=pltpu.CompilerParams(use_tc_tiling_on_sc=False),
        scratch_types=[
            pltpu.VMEM((n_chunks, CH), jnp.int32),
            pltpu.VMEM((n_chunks, CH), jnp.int32),
            pltpu.VMEM((CH, dh), jnp.float32),
            pltpu.VMEM((CH, dh), jnp.float32),
            pltpu.VMEM((CH, dh), jnp.float32),
            pltpu.VMEM((CH, dh), jnp.float32),
            pltpu.VMEM_SHARED((n, dh), jnp.float32),
            pltpu.SemaphoreType.DMA,
            pltpu.SemaphoreType.DMA,
            pltpu.SemaphoreType.DMA,
            pltpu.SemaphoreType.DMA,
        ],
    )
    def sc_conv(x_hbm, coeff_hbm, send_hbm, recv_hbm, out_hbm,
                sidx_v, ridx_v, rows0, rows1, coef0, coef1, agg_sh,
                gs0, gs1, cs0, cs1):
        cid = lax.axis_index("c")
        sid = lax.axis_index("s")
        wid = cid * NS + sid
        base = wid * per_tile
        nrc = jnp.clip((e_real - base + CH - 1) // CH, 0, n_chunks)

        def zrow(j, c2):
            for k in range(dh // L):
                rows0[j, pl.ds(k * L, L)] = jnp.zeros((L,), jnp.float32)
            return c2
        lax.fori_loop(0, CH, zrow, 0)
        for zo, zs in zblocks:
            pltpu.sync_copy(rows0.at[pl.ds(0, zs)],
                            agg_sh.at[pl.ds(sid * rows_per_sid + zo, zs)])
        plsc.subcore_barrier()

        pltpu.sync_copy(send_hbm.at[pl.ds(wid * n_chunks, n_chunks), :], sidx_v)
        pltpu.sync_copy(recv_hbm.at[pl.ds(wid * n_chunks, n_chunks), :], ridx_v)

        def issue(t, rows, coef, gsem, csem):
            off = base + t * CH
            pltpu.async_copy(coeff_hbm.at[pl.ds(off, CH), pl.ds(col, dh)], coef, csem)
            pltpu.async_copy(x_hbm.at[sidx_v.at[t]], rows, gsem)

        def process(t, rows, coef, gsem, csem):
            pltpu.make_async_copy(coeff_hbm.at[pl.ds(0, CH), pl.ds(col, dh)], coef, csem).wait()
            pltpu.make_async_copy(x_hbm.at[sidx_v.at[0]], rows, gsem).wait()

            def row(j, c2):
                for k in range(dh // L):
                    sl = pl.ds(k * L, L)
                    rows[j, sl] = rows[j, sl] * coef[j, sl]
                return c2
            lax.fori_loop(0, CH, row, 0)
            pltpu.sync_copy(rows, agg_sh.at[ridx_v.at[t]], add=True)

        @pl.when(nrc > 0)
        def _():
            issue(0, rows0, coef0, gs0, cs0)

        def pair(u, carry):
            c = 2 * u

            @pl.when(c + 1 < nrc)
            def _():
                issue(c + 1, rows1, coef1, gs1, cs1)

            @pl.when(c < nrc)
            def _():
                process(c, rows0, coef0, gs0, cs0)

            @pl.when(c + 2 < nrc)
            def _():
                issue(c + 2, rows0, coef0, gs0, cs0)

            @pl.when(c + 1 < nrc)
            def _():
                process(c + 1, rows1, coef1, gs1, cs1)
            return carry

        lax.fori_loop(0, (n_chunks + 1) // 2, pair, 0)
        plsc.subcore_barrier()

        pltpu.sync_copy(agg_sh.at[pl.ds(sid * rows_per_sid, rows_per_sid)],
                        out_hbm.at[cid, pl.ds(sid * rows_per_sid, rows_per_sid)])

    return sc_conv


def _out_body(p0_ref, p1_ref, nf_ref, sp_ref, wout_ref, wsc_ref, o_ref, *,
              inv_sqrt_neigh, s, dh):
    agg0 = (p0_ref[0] + p0_ref[1]) * inv_sqrt_neigh
    agg1 = (p1_ref[0] + p1_ref[1]) * inv_sqrt_neigh
    out = (jnp.dot(agg0, wout_ref[pl.ds(0, dh), :], preferred_element_type=jnp.float32)
           + jnp.dot(agg1, wout_ref[pl.ds(dh, dh), :], preferred_element_type=jnp.float32))
    nf = nf_ref[...]
    sp = sp_ref[...]
    sc = jnp.zeros_like(out)
    for k in range(s):
        cand = jnp.dot(nf, wsc_ref[k], preferred_element_type=jnp.float32)
        sc = jnp.where(sp == k, cand, sc)
    z = out + sc
    o_ref[...] = z * jax.nn.sigmoid(z)


def _finalize(p0, p1, nf, species2d, W_out, W_sc, inv_sqrt_neigh, bn):
    n, d = nf.shape
    dh = d // 2
    s = W_sc.shape[0]
    body = functools.partial(_out_body, inv_sqrt_neigh=inv_sqrt_neigh, s=s, dh=dh)
    return pl.pallas_call(
        body,
        grid=(n // bn,),
        in_specs=[
            pl.BlockSpec((NC, bn, dh), lambda i: (0, i, 0)),
            pl.BlockSpec((NC, bn, dh), lambda i: (0, i, 0)),
            pl.BlockSpec((bn, d), lambda i: (i, 0)),
            pl.BlockSpec((bn, 1), lambda i: (i, 0)),
            pl.BlockSpec((d, d), lambda i: (0, 0)),
            pl.BlockSpec((s, d, d), lambda i: (0, 0, 0)),
        ],
        out_specs=pl.BlockSpec((bn, d), lambda i: (i, 0)),
        out_shape=jax.ShapeDtypeStruct((n, d), jnp.float32),
    )(p0, p1, nf, species2d, W_out, W_sc)


def kernel(node_features, edge_attributes, radial_embeddings, senders,
           receivers, species, W_in, W_r0, W_r1, W_attr, W_out, W_sc):
    n, d = node_features.shape
    e = senders.shape[0]
    avg_neigh = 32.0

    nw = NC * NS
    gran = nw * CH
    ep = ((e + gran - 1) // gran) * gran
    pad = ep - e
    send_p = jnp.pad(senders.astype(jnp.int32), (0, pad)).reshape(ep // CH, CH)
    recv_p = jnp.pad(receivers.astype(jnp.int32), (0, pad)).reshape(ep // CH, CH)

    coeff = _coeff(radial_embeddings, edge_attributes, W_r0, W_r1, W_attr, be=2560)
    x0, x1 = _in_linear(node_features, W_in, bn=1000)

    ngran = NS * 8
    npad = ((n + ngran - 1) // ngran) * ngran
    p0 = _make_sc_conv(npad, d // 2, ep, e, 0)(x0, coeff, send_p, recv_p)
    p1 = _make_sc_conv(npad, d // 2, ep, e, d // 2)(x1, coeff, send_p, recv_p)

    species2d = species.astype(jnp.int32).reshape(n, 1)
    return _finalize(p0, p1, node_features, species2d, W_out, W_sc,
                     1.0 / (avg_neigh ** 0.5), bn=1000)


# speedup vs baseline: 5.5318x; 1.1359x over previous
import functools

import jax
import jax.numpy as jnp
from jax import lax
from jax.experimental import pallas as pl
from jax.experimental.pallas import tpu as pltpu
from jax.experimental.pallas import tpu_sc as plsc

NC = 2
NS = 16
L = 16
CH = 128


def _coeff_body(rt_ref, at_ref, wr0t_ref, wr1t_ref, wattr_ref, o_ref, *, cb):
    ht = jax.nn.swish(jnp.dot(wr0t_ref[...], rt_ref[...],
                              preferred_element_type=jnp.float32))
    be = rt_ref.shape[1]
    wattr = wattr_ref[...]
    wr1t = wr1t_ref[...]
    for k in range(be // cb):
        lo, hi = k * cb, (k + 1) * cb
        radt = jnp.dot(wr1t, ht[:, lo:hi], preferred_element_type=jnp.float32)
        amt = jnp.dot(wattr, at_ref[:, lo:hi], preferred_element_type=jnp.float32)
        o_ref[lo:hi, :] = (radt * amt).T


def _coeff(radial, attrs, W_r0, W_r1, W_attr, be):
    e, r_dim = radial.shape
    de = attrs.shape[1]
    d = W_r1.shape[1]
    rt = radial.T
    at = attrs.T
    return pl.pallas_call(
        functools.partial(_coeff_body, cb=256),
        grid=(e // be,),
        in_specs=[
            pl.BlockSpec((r_dim, be), lambda i: (0, i)),
            pl.BlockSpec((de, be), lambda i: (0, i)),
            pl.BlockSpec((r_dim, r_dim), lambda i: (0, 0)),
            pl.BlockSpec((d, r_dim), lambda i: (0, 0)),
            pl.BlockSpec((d, de), lambda i: (0, 0)),
        ],
        out_specs=pl.BlockSpec((be, d), lambda i: (i, 0)),
        out_shape=jax.ShapeDtypeStruct((e, d), jnp.float32),
    )(rt, at, W_r0.T, W_r1.T, W_attr)


def _matmul_body(x_ref, w_ref, o0_ref, o1_ref):
    res = jnp.dot(x_ref[...], w_ref[...], preferred_element_type=jnp.float32)
    half = res.shape[1] // 2
    o0_ref[...] = res[:, :half]
    o1_ref[...] = res[:, half:]


def _in_linear(nf, W_in, bn):
    n, d = nf.shape
    dh = d // 2
    return pl.pallas_call(
        _matmul_body,
        grid=(n // bn,),
        in_specs=[
            pl.BlockSpec((bn, d), lambda i: (i, 0)),
            pl.BlockSpec((d, d), lambda i: (0, 0)),
        ],
        out_specs=[pl.BlockSpec((bn, dh), lambda i: (i, 0)),
                   pl.BlockSpec((bn, dh), lambda i: (i, 0))],
        out_shape=[jax.ShapeDtypeStruct((n, dh), jnp.float32),
                   jax.ShapeDtypeStruct((n, dh), jnp.float32)],
    )(nf, W_in)


def _make_sc_conv(n, dh, ep, e_real, col):
    nw = NC * NS
    per_tile = ep // nw
    n_chunks = per_tile // CH
    rows_per_sid = n // NS
    zblocks = [(o, min(CH, rows_per_sid - o)) for o in range(0, rows_per_sid, CH)]
    mesh = plsc.VectorSubcoreMesh(core_axis_name="c", subcore_axis_name="s")

    @functools.partial(
        pl.kernel,
        out_type=jax.ShapeDtypeStruct((NC, n, dh), jnp.float32),
        mesh=mesh,
        compiler_params=pltpu.CompilerParams(use_tc_tiling_on_sc=False),
        scratch_types=[
            pltpu.VMEM((n_chunks, CH), jnp.int32),
            pltpu.VMEM((n_chunks, CH), jnp.int32),
            pltpu.VMEM((CH, dh), jnp.float32),
            pltpu.VMEM((CH, dh), jnp.float32),
            pltpu.VMEM((CH, dh), jnp.float32),
            pltpu.VMEM((CH, dh), jnp.float32),
            pltpu.VMEM_SHARED((n, dh), jnp.float32),
            pltpu.SemaphoreType.DMA,
            pltpu.SemaphoreType.DMA,
            pltpu.SemaphoreType.DMA,
            pltpu.SemaphoreType.DMA,
        ],
    )
    def sc_conv(x_hbm, coeff_hbm, send_hbm, recv_hbm, out_hbm,
                sidx_v, ridx_v, rows0, rows1, coef0, coef1, agg_sh,
                gs0, gs1, cs0, cs1):
        cid = lax.axis_index("c")
        sid = lax.axis_index("s")
        wid = cid * NS + sid
        base = wid * per_tile
        nrc = jnp.clip((e_real - base + CH - 1) // CH, 0, n_chunks)

        def zrow(j, c2):
            for k in range(dh // L):
                rows0[j, pl.ds(k * L, L)] = jnp.zeros((L,), jnp.float32)
            return c2
        lax.fori_loop(0, CH, zrow, 0)
        for zo, zs in zblocks:
            pltpu.sync_copy(rows0.at[pl.ds(0, zs)],
                            agg_sh.at[pl.ds(sid * rows_per_sid + zo, zs)])
        plsc.subcore_barrier()

        pltpu.sync_copy(send_hbm.at[pl.ds(wid * n_chunks, n_chunks), :], sidx_v)
        pltpu.sync_copy(recv_hbm.at[pl.ds(wid * n_chunks, n_chunks), :], ridx_v)

        def issue(t, rows, coef, gsem, csem):
            off = base + t * CH
            pltpu.async_copy(coeff_hbm.at[pl.ds(off, CH), pl.ds(col, dh)], coef, csem)
            pltpu.async_copy(x_hbm.at[sidx_v.at[t]], rows, gsem)

        def process(t, rows, coef, gsem, csem):
            pltpu.make_async_copy(coeff_hbm.at[pl.ds(0, CH), pl.ds(col, dh)], coef, csem).wait()
            pltpu.make_async_copy(x_hbm.at[sidx_v.at[0]], rows, gsem).wait()

            def row(j, c2):
                for k in range(dh // L):
                    sl = pl.ds(k * L, L)
                    rows[j, sl] = rows[j, sl] * coef[j, sl]
                return c2
            lax.fori_loop(0, CH, row, 0)
            pltpu.sync_copy(rows, agg_sh.at[ridx_v.at[t]], add=True)

        @pl.when(nrc > 0)
        def _():
            issue(0, rows0, coef0, gs0, cs0)

        def pair(u, carry):
            c = 2 * u

            @pl.when(c + 1 < nrc)
            def _():
                issue(c + 1, rows1, coef1, gs1, cs1)

            @pl.when(c < nrc)
            def _():
                process(c, rows0, coef0, gs0, cs0)

            @pl.when(c + 2 < nrc)
            def _():
                issue(c + 2, rows0, coef0, gs0, cs0)

            @pl.when(c + 1 < nrc)
            def _():
                process(c + 1, rows1, coef1, gs1, cs1)
            return carry

        lax.fori_loop(0, (n_chunks + 1) // 2, pair, 0)
        plsc.subcore_barrier()

        pltpu.sync_copy(agg_sh.at[pl.ds(sid * rows_per_sid, rows_per_sid)],
                        out_hbm.at[cid, pl.ds(sid * rows_per_sid, rows_per_sid)])

    return sc_conv


def _out_body(p0_ref, p1_ref, nf_ref, sp_ref, wout_ref, wsc_ref, o_ref, *,
              inv_sqrt_neigh, s, dh):
    agg0 = (p0_ref[0] + p0_ref[1]) * inv_sqrt_neigh
    agg1 = (p1_ref[0] + p1_ref[1]) * inv_sqrt_neigh
    out = (jnp.dot(agg0, wout_ref[pl.ds(0, dh), :], preferred_element_type=jnp.float32)
           + jnp.dot(agg1, wout_ref[pl.ds(dh, dh), :], preferred_element_type=jnp.float32))
    nf = nf_ref[...]
    sp = sp_ref[...]
    sc = jnp.zeros_like(out)
    for k in range(s):
        cand = jnp.dot(nf, wsc_ref[k], preferred_element_type=jnp.float32)
        sc = jnp.where(sp == k, cand, sc)
    z = out + sc
    o_ref[...] = z * jax.nn.sigmoid(z)


def _finalize(p0, p1, nf, species2d, W_out, W_sc, inv_sqrt_neigh, bn):
    n, d = nf.shape
    dh = d // 2
    s = W_sc.shape[0]
    body = functools.partial(_out_body, inv_sqrt_neigh=inv_sqrt_neigh, s=s, dh=dh)
    return pl.pallas_call(
        body,
        grid=(n // bn,),
        in_specs=[
            pl.BlockSpec((NC, bn, dh), lambda i: (0, i, 0)),
            pl.BlockSpec((NC, bn, dh), lambda i: (0, i, 0)),
            pl.BlockSpec((bn, d), lambda i: (i, 0)),
            pl.BlockSpec((bn, 1), lambda i: (i, 0)),
            pl.BlockSpec((d, d), lambda i: (0, 0)),
            pl.BlockSpec((s, d, d), lambda i: (0, 0, 0)),
        ],
        out_specs=pl.BlockSpec((bn, d), lambda i: (i, 0)),
        out_shape=jax.ShapeDtypeStruct((n, d), jnp.float32),
    )(p0, p1, nf, species2d, W_out, W_sc)


def kernel(node_features, edge_attributes, radial_embeddings, senders,
           receivers, species, W_in, W_r0, W_r1, W_attr, W_out, W_sc):
    n, d = node_features.shape
    e = senders.shape[0]
    avg_neigh = 32.0

    nw = NC * NS
    gran = nw * CH
    ep = ((e + gran - 1) // gran) * gran
    pad = ep - e
    send_p = jnp.pad(senders.astype(jnp.int32), (0, pad)).reshape(ep // CH, CH)
    recv_p = jnp.pad(receivers.astype(jnp.int32), (0, pad)).reshape(ep // CH, CH)

    coeff = _coeff(radial_embeddings, edge_attributes, W_r0, W_r1, W_attr, be=6400)
    x0, x1 = _in_linear(node_features, W_in, bn=1000)

    ngran = NS * 8
    npad = ((n + ngran - 1) // ngran) * ngran
    p0 = _make_sc_conv(npad, d // 2, ep, e, 0)(x0, coeff, send_p, recv_p)
    p1 = _make_sc_conv(npad, d // 2, ep, e, d // 2)(x1, coeff, send_p, recv_p)

    species2d = species.astype(jnp.int32).reshape(n, 1)
    return _finalize(p0, p1, node_features, species2d, W_out, W_sc,
                     1.0 / (avg_neigh ** 0.5), bn=1000)
